# initial kernel scaffold (unmeasured)
import jax
import jax.numpy as jnp
from jax import lax
from jax.experimental import pallas as pl
from jax.experimental.pallas import tpu as pltpu


def kernel(
    x,
):
    def body(*refs):
        pass

    out_shape = jax.ShapeDtypeStruct(..., jnp.float32)
    return pl.pallas_call(body, out_shape=out_shape)(...)


# baseline (device time: 119492 ns/iter reference)
import jax
import jax.numpy as jnp
from jax import lax
from jax.experimental import pallas as pl
from jax.experimental.pallas import tpu as pltpu

N_DEV = 16


def kernel(x):
    m_per, n = x.shape

    def body(x_ref, out_ref, send_sems, recv_sems):
        my = lax.axis_index("i")
        left = lax.rem(my - 1 + N_DEV, N_DEV)
        right = lax.rem(my + 1, N_DEV)

        barrier_sem = pltpu.get_barrier_semaphore()
        for nbr in (left, right):
            pl.semaphore_signal(
                barrier_sem, inc=1,
                device_id=(nbr,), device_id_type=pl.DeviceIdType.MESH,
            )
        pl.semaphore_wait(barrier_sem, 2)

        out_ref[pl.ds(my * m_per, m_per), :] = x_ref[...].astype(out_ref.dtype)

        for h in range(N_DEV - 1):
            origin = lax.rem(my - h + N_DEV, N_DEV)
            recv_origin = lax.rem(my - h - 1 + N_DEV, N_DEV)
            rdma = pltpu.make_async_remote_copy(
                src_ref=out_ref.at[pl.ds(origin * m_per, m_per)],
                dst_ref=out_ref.at[pl.ds(origin * m_per, m_per)],
                send_sem=send_sems.at[h],
                recv_sem=recv_sems.at[h],
                device_id=(right,),
                device_id_type=pl.DeviceIdType.MESH,
            )
            rdma.start()
            rdma.wait()
            del recv_origin

    return pl.pallas_call(
        body,
        out_shape=jax.ShapeDtypeStruct((N_DEV * m_per, n), jnp.bfloat16),
        in_specs=[pl.BlockSpec(memory_space=pltpu.VMEM)],
        out_specs=pl.BlockSpec(memory_space=pltpu.VMEM),
        scratch_shapes=[
            pltpu.SemaphoreType.DMA((N_DEV - 1,)),
            pltpu.SemaphoreType.DMA((N_DEV - 1,)),
        ],
        compiler_params=pltpu.CompilerParams(collective_id=0),
    )(x)


# device time: 54800 ns/iter; 2.1805x vs baseline; 2.1805x over previous
import jax
import jax.numpy as jnp
from jax import lax
from jax.experimental import pallas as pl
from jax.experimental.pallas import tpu as pltpu

N_DEV = 16
N_FULL_HOPS = 7
N_SEG = 2


def kernel(x):
    m_per, n = x.shape
    seg = m_per // N_SEG
    half = m_per // 2

    def body(x_ref, out_ref, fs_sems, fr_sems, bs_sems, br_sems):
        my = lax.axis_index("i")
        left = lax.rem(my - 1 + N_DEV, N_DEV)
        right = lax.rem(my + 1, N_DEV)

        barrier_sem = pltpu.get_barrier_semaphore()
        for nbr in (left, right):
            pl.semaphore_signal(
                barrier_sem, inc=1,
                device_id=(nbr,), device_id_type=pl.DeviceIdType.MESH,
            )
        pl.semaphore_wait(barrier_sem, 2)

        out_ref[pl.ds(my * m_per, m_per), :] = x_ref[...].astype(out_ref.dtype)

        def copy(origin, row0, nrows, sems_pair, h, s, dev):
            sl = pl.ds(origin * m_per + row0, nrows)
            return pltpu.make_async_remote_copy(
                src_ref=out_ref.at[sl],
                dst_ref=out_ref.at[sl],
                send_sem=sems_pair[0].at[h, s],
                recv_sem=sems_pair[1].at[h, s],
                device_id=(dev,),
                device_id_type=pl.DeviceIdType.MESH,
            )

        fwd = (fs_sems, fr_sems)
        bwd = (bs_sems, br_sems)

        def fwd_send(h, s):
            return copy(lax.rem(my - h + N_DEV, N_DEV), s * seg, seg,
                        fwd, h, s, right)

        def fwd_recv(h, s):
            return copy(lax.rem(my - 1 - h + N_DEV, N_DEV), s * seg, seg,
                        fwd, h, s, left)

        def bwd_send(h, s):
            return copy(lax.rem(my + h, N_DEV), s * seg, seg, bwd, h, s, left)

        def bwd_recv(h, s):
            return copy(lax.rem(my + 1 + h, N_DEV), s * seg, seg,
                        bwd, h, s, right)

        sends = []

        def start(d):
            d.start()
            sends.append(d)

        for s in range(N_SEG):
            start(fwd_send(0, s))
            start(bwd_send(0, s))

        for h in range(1, N_FULL_HOPS):
            for s in range(N_SEG):
                fwd_recv(h - 1, s).wait_recv()
                start(fwd_send(h, s))
                bwd_recv(h - 1, s).wait_recv()
                start(bwd_send(h, s))

        fwd_recv(N_FULL_HOPS - 1, 0).wait_recv()
        start(copy(lax.rem(my - N_FULL_HOPS + N_DEV, N_DEV), 0, half,
                   fwd, N_FULL_HOPS, 0, right))
        bwd_recv(N_FULL_HOPS - 1, 1).wait_recv()
        start(copy(lax.rem(my + N_FULL_HOPS, N_DEV), half, half,
                   bwd, N_FULL_HOPS, 0, left))

        fwd_recv(N_FULL_HOPS - 1, 1).wait_recv()
        bwd_recv(N_FULL_HOPS - 1, 0).wait_recv()
        anti = lax.rem(my + N_DEV // 2, N_DEV)
        copy(anti, 0, half, fwd, N_FULL_HOPS, 0, left).wait_recv()
        copy(anti, half, half, bwd, N_FULL_HOPS, 0, right).wait_recv()

        for d in sends:
            d.wait_send()

    return pl.pallas_call(
        body,
        out_shape=jax.ShapeDtypeStruct((N_DEV * m_per, n), jnp.bfloat16),
        in_specs=[pl.BlockSpec(memory_space=pltpu.VMEM)],
        out_specs=pl.BlockSpec(memory_space=pltpu.VMEM),
        scratch_shapes=[
            pltpu.SemaphoreType.DMA((N_FULL_HOPS + 1, N_SEG)),
            pltpu.SemaphoreType.DMA((N_FULL_HOPS + 1, N_SEG)),
            pltpu.SemaphoreType.DMA((N_FULL_HOPS + 1, N_SEG)),
            pltpu.SemaphoreType.DMA((N_FULL_HOPS + 1, N_SEG)),
        ],
        compiler_params=pltpu.CompilerParams(collective_id=0),
    )(x)
